# baseline (speedup 1.0000x reference)
import jax
import jax.numpy as jnp
import numpy as np
from jax.experimental import pallas as pl
from jax.experimental.pallas import tpu as pltpu

_IN, _H, _L, _S = 512, 512, 2, 3
_B, _T = 64, 512
_PENALTY = 0.7
_LOG_PENALTY = float(np.log(np.float32(_PENALTY)))
_TC = 32
_UNROLL = 1
_G3 = 3 * _H


def _sigmoid(x):
    return 1.0 / (1.0 + jnp.exp(-x))


def _body(xb_ref, wlw_ref, wsel_ref, bsel_ref, blw_ref,
          wih0_ref, whh0_ref, bih0_ref, bhh0_ref,
          wih1_ref, whh1_ref, bih1_ref, bhh1_ref,
          out_ref, h_ref, lp_ref):
    t_base = pl.program_id(0) * _TC
    f32, bf16 = jnp.float32, jnp.bfloat16

    @pl.when(t_base == 0)
    def _init():
        h_ref[...] = jnp.zeros((2 * _B, _H), f32)
        lp_ref[0] = 0.0
        lp_ref[1] = 0.0
        lp_ref[2] = 0.0

    def step(t, tl, hc):
        h0, h1 = hc
        x_t = xb_ref[pl.ds(tl, 1)].reshape(_B, _IN)

        lgx = jnp.dot(x_t, wsel_ref[_H:], preferred_element_type=f32)

        h0b = h0.astype(bf16)
        h1b = h1.astype(bf16)
        hb2 = jnp.concatenate([h0b, h1b], axis=0)

        energy = jnp.dot(hb2, wlw_ref[...], preferred_element_type=f32)
        le = jnp.sum(energy + blw_ref[...], axis=-1, keepdims=True)
        prod = le.astype(bf16) * hb2
        sh = prod[:_B] + prod[_B:]
        logits = (jnp.dot(sh, wsel_ref[:_H], preferred_element_type=f32)
                  + lgx + bsel_ref[...])
        slog = jnp.sum(logits, axis=0, keepdims=True)

        s0 = slog[0, 0] + lp_ref[0]
        s1 = slog[0, 1] + lp_ref[1]
        s2 = slog[0, 2] + lp_ref[2]
        cur = jnp.where((s0 >= s1) & (s0 >= s2), 0,
                        jnp.where(s1 >= s2, 1, 2)).astype(jnp.int32)
        cur = jnp.where(t == 0, 0, cur)

        l0 = lp_ref[0] + jnp.where(cur == 0, _LOG_PENALTY, 0.0)
        l1 = lp_ref[1] + jnp.where(cur == 1, _LOG_PENALTY, 0.0)
        l2 = lp_ref[2] + jnp.where(cur == 2, _LOG_PENALTY, 0.0)
        m = jnp.maximum(l0, jnp.maximum(l1, l2))
        lp_ref[0] = l0 - m
        lp_ref[1] = l1 - m
        lp_ref[2] = l2 - m

        w = lambda ref: ref[pl.ds(cur, 1)].reshape(_IN, _G3)
        b = lambda ref: ref[pl.ds(cur, 1)].reshape(1, _G3)

        def gates(gi, gh, h_prev):
            r = _sigmoid(gi[:, :_H] + gh[:, :_H])
            z = _sigmoid(gi[:, _H:2 * _H] + gh[:, _H:2 * _H])
            n = jnp.tanh(gi[:, 2 * _H:] + r * gh[:, 2 * _H:])
            return (1.0 - z) * n + z * h_prev

        gi0 = jnp.dot(x_t, w(wih0_ref), preferred_element_type=f32) + b(bih0_ref)
        gh0 = jnp.dot(h0b, w(whh0_ref), preferred_element_type=f32) + b(bhh0_ref)
        gh1 = jnp.dot(h1b, w(whh1_ref), preferred_element_type=f32) + b(bhh1_ref)
        h0n = gates(gi0, gh0, h0)
        gi1 = jnp.dot(h0n.astype(bf16), w(wih1_ref),
                      preferred_element_type=f32) + b(bih1_ref)
        h1n = gates(gi1, gh1, h1)

        out_ref[pl.ds(tl, 1)] = h1n.reshape(1, _B, _H)
        return (h0n, h1n)

    def pair(u, hc):
        tl = u * _UNROLL
        for k in range(_UNROLL):
            hc = step(t_base + tl + k, tl + k, hc)
        return hc

    h0f, h1f = jax.lax.fori_loop(0, _TC // _UNROLL, pair,
                                 (h_ref[:_B], h_ref[_B:]))
    h_ref[:_B] = h0f
    h_ref[_B:] = h1f


def kernel(x, Wih_first, Wih_rest, Whh, bih, bhh, Wlw, blw, Wsel, bsel):
    f32, bf16 = jnp.float32, jnp.bfloat16
    xb = jnp.swapaxes(x, 0, 1).astype(bf16)
    wlw_t = Wlw.T.astype(bf16)
    wsel_t = jnp.zeros((_H + _IN, 128), f32).at[:, :_S].set(Wsel.T).astype(bf16)
    bsel_p = jnp.zeros((1, 128), f32).at[0, :_S].set(bsel)
    blw_r = blw.reshape(1, _H)
    wih0 = Wih_first.transpose(0, 2, 1).astype(bf16)
    wih1 = Wih_rest[:, 0].transpose(0, 2, 1).astype(bf16)
    whh0 = Whh[:, 0].transpose(0, 2, 1).astype(bf16)
    whh1 = Whh[:, 1].transpose(0, 2, 1).astype(bf16)
    bih0, bih1 = bih[:, 0][:, None, :], bih[:, 1][:, None, :]
    bhh0, bhh1 = bhh[:, 0][:, None, :], bhh[:, 1][:, None, :]

    full = lambda a: pl.BlockSpec(a.shape, lambda i: (0,) * a.ndim)
    outputs = pl.pallas_call(
        _body,
        grid=(_T // _TC,),
        in_specs=[pl.BlockSpec((_TC, _B, _IN), lambda i: (i, 0, 0))]
        + [full(a) for a in (wlw_t, wsel_t, bsel_p, blw_r,
                             wih0, whh0, bih0, bhh0, wih1, whh1, bih1, bhh1)],
        out_specs=pl.BlockSpec((_TC, _B, _H), lambda i: (i, 0, 0)),
        out_shape=jax.ShapeDtypeStruct((_T, _B, _H), f32),
        scratch_shapes=[pltpu.VMEM((2 * _B, _H), f32),
                        pltpu.SMEM((8,), f32)],
        compiler_params=pltpu.CompilerParams(
            dimension_semantics=("arbitrary",),
            vmem_limit_bytes=64 * 1024 * 1024,
        ),
    )(xb, wlw_t, wsel_t, bsel_p, blw_r,
      wih0, whh0, bih0, bhh0, wih1, whh1, bih1, bhh1)
    return outputs, outputs[-1]
